# baseline (speedup 1.0000x reference)
import jax
import jax.numpy as jnp
from jax import lax
from jax.experimental import pallas as pl
from jax.experimental.pallas import tpu as pltpu
from jax.experimental.pallas import tpu_sc as plsc

B, N, M, K = 8, 32768, 8192, 128
NC, NS, L = 2, 16, 16
NW = NC * NS
QPT = (B * N) // NW
GU = 8
CHUNKS = QPT // (GU * L)


def _nn_body(verts_hbm, coll_hbm, cv_hbm, out_hbm,
             vbuf, collbuf, cvbuf, sxbuf, sybuf, szbuf, obuf):
    wid = lax.axis_index("s") * NC + lax.axis_index("c")
    b = wid // (N // QPT)
    qoff = wid * QPT

    pltpu.sync_copy(cv_hbm, cvbuf)
    pltpu.sync_copy(verts_hbm.at[pl.ds(qoff * 3, QPT * 3)], vbuf)
    pltpu.sync_copy(coll_hbm.at[pl.ds(b * (M * 3), M * 3)], collbuf)

    for kk in range(K // L):
        idx3 = cvbuf[pl.ds(kk * L, L)] * 3
        sxv = plsc.load_gather(collbuf, [idx3])
        syv = plsc.load_gather(collbuf, [idx3 + 1])
        szv = plsc.load_gather(collbuf, [idx3 + 2])
        for j in range(L):
            base = (kk * L + j) * L
            sxbuf[pl.ds(base, L)] = jnp.full((L,), sxv[j], jnp.float32)
            sybuf[pl.ds(base, L)] = jnp.full((L,), syv[j], jnp.float32)
            szbuf[pl.ds(base, L)] = jnp.full((L,), szv[j], jnp.float32)

    lanes = lax.iota(jnp.int32, L)
    bvec = jnp.full((L,), b, jnp.int32)
    inf = jnp.full((L,), jnp.inf, jnp.float32)
    zero = jnp.zeros((L,), jnp.int32)

    def chunk_body(c, carry):
        base = c * (GU * L)
        vx, vy, vz = [], [], []
        for g in range(GU):
            a3 = (base + g * L + lanes) * 3
            vx.append(plsc.load_gather(vbuf, [a3]))
            vy.append(plsc.load_gather(vbuf, [a3 + 1]))
            vz.append(plsc.load_gather(vbuf, [a3 + 2]))

        def k_body(k, bc):
            best, bidx = bc
            o = k * L
            kv = jnp.full((L,), k, jnp.int32)
            sx = sxbuf[pl.ds(o, L)]
            sy = sybuf[pl.ds(o, L)]
            sz = szbuf[pl.ds(o, L)]
            nbest, nbidx = [], []
            for g in range(GU):
                dx = vx[g] - sx
                dy = vy[g] - sy
                dz = vz[g] - sz
                d2 = (dx * dx + dy * dy) + dz * dz
                m = d2 < best[g]
                nbest.append(jnp.where(m, d2, best[g]))
                nbidx.append(jnp.where(m, kv, bidx[g]))
            return nbest, nbidx

        best, bidx = lax.fori_loop(0, K, k_body, ([inf] * GU, [zero] * GU))
        for g in range(GU):
            q2 = (base + g * L + lanes) * 2
            plsc.store_scatter(obuf, [q2], bvec)
            plsc.store_scatter(obuf, [q2 + 1], bidx[g])
        return carry

    lax.fori_loop(0, CHUNKS, chunk_body, 0)
    pltpu.sync_copy(obuf, out_hbm.at[pl.ds(qoff * 2, QPT * 2)])


def kernel(vertices, collider, collision_vertices):
    mesh = plsc.VectorSubcoreMesh(core_axis_name="c", subcore_axis_name="s")
    run = pl.kernel(
        _nn_body,
        out_type=jax.ShapeDtypeStruct((B * N * 2,), jnp.int32),
        mesh=mesh,
        compiler_params=pltpu.CompilerParams(needs_layout_passes=False),
        scratch_types=[
            pltpu.VMEM((QPT * 3,), jnp.float32),
            pltpu.VMEM((M * 3,), jnp.float32),
            pltpu.VMEM((K,), jnp.int32),
            pltpu.VMEM((K * L,), jnp.float32),
            pltpu.VMEM((K * L,), jnp.float32),
            pltpu.VMEM((K * L,), jnp.float32),
            pltpu.VMEM((QPT * 2,), jnp.int32),
        ],
    )
    out = run(vertices.reshape(-1), collider.reshape(-1), collision_vertices)
    return out.reshape(B, N, 2)


# speedup vs baseline: 3.2994x; 3.2994x over previous
import jax
import jax.numpy as jnp
from jax import lax
from jax.experimental import pallas as pl
from jax.experimental.pallas import tpu as pltpu
from jax.experimental.pallas import tpu_sc as plsc

B, N, M, K = 8, 32768, 8192, 128
NC, NS, L = 2, 16, 16
NW = NC * NS
QPT = (B * N) // NW
GU = 8
CHUNKS = QPT // (GU * L)
PLANE_V = B * N
PLANE_C = B * M


def _nn_body(verts_hbm, coll_hbm, cv_hbm, out_hbm,
             vxb, vyb, vzb, cvbuf, fidx, selb, srep, obuf, sem):
    wid = lax.axis_index("s") * NC + lax.axis_index("c")

    pltpu.sync_copy(cv_hbm, cvbuf)
    for c in range(3):
        dst = (vxb, vyb, vzb)[c]
        pltpu.sync_copy(verts_hbm.at[pl.ds(c * PLANE_V + wid * QPT, QPT)], dst)

    def fidx_body(r, carry):
        c = r // 8
        b = r - c * 8
        cb = c * PLANE_C + b * 128
        for jj in range(K // L):
            m = cvbuf[pl.ds(jj * L, L)]
            off = ((m >> 7) << 10) + (m & 127) + cb
            fidx[pl.ds(r * K + jj * L, L)] = off
        return carry

    lax.fori_loop(0, 24, fidx_body, 0)

    copies = [
        pltpu.async_copy(coll_hbm.at[fidx.at[pl.ds(r * K, K)]],
                         selb.at[pl.ds(r * K, K)], sem)
        for r in range(24)
    ]
    for cp in copies:
        cp.wait()

    def rep_body(r, carry):
        for jj in range(K // L):
            v16 = selb[pl.ds(r * K + jj * L, L)]
            for l in range(L):
                srep[pl.ds((r * K + jj * L + l) * L, L)] = (
                    jnp.full((L,), v16[l], jnp.float32))
        return carry

    lax.fori_loop(0, 24, rep_body, 0)

    lanes = lax.iota(jnp.int32, L)
    inf = jnp.full((L,), jnp.inf, jnp.float32)
    zero = jnp.zeros((L,), jnp.int32)

    def chunk_body(c2, carry):
        b = c2 & 7
        p0 = c2 * 128
        vx, vy, vz = [], [], []
        for g in range(GU):
            vx.append(vxb[pl.ds(p0 + g * L, L)])
            vy.append(vyb[pl.ds(p0 + g * L, L)])
            vz.append(vzb[pl.ds(p0 + g * L, L)])

        sbase = b * (K * L)

        def k_body(k, bc):
            best, bidx = bc
            o = k * L
            kv = jnp.full((L,), k, jnp.int32)
            sx = srep[pl.ds(sbase + o, L)]
            sy = srep[pl.ds(PLANE_S + sbase + o, L)]
            sz = srep[pl.ds(2 * PLANE_S + sbase + o, L)]
            nbest, nbidx = [], []
            for g in range(GU):
                dx = vx[g] - sx
                dy = vy[g] - sy
                dz = vz[g] - sz
                d2 = (dx * dx + dy * dy) + dz * dz
                mlt = d2 < best[g]
                nbest.append(jnp.where(mlt, d2, best[g]))
                nbidx.append(jnp.where(mlt, kv, bidx[g]))
            return nbest, nbidx

        best, bidx = lax.fori_loop(0, K, k_body, ([inf] * GU, [zero] * GU))

        q = c2 >> 3
        obase = b * 2048 + q * 256
        bvec = jnp.full((L,), b, jnp.int32)
        for g in range(GU):
            obuf[pl.ds(obase + g * L, L)] = bvec
            obuf[pl.ds(obase + 128 + g * L, L)] = bidx[g]
        return carry

    lax.fori_loop(0, CHUNKS, chunk_body, 0)

    for b in range(B):
        pltpu.sync_copy(obuf.at[pl.ds(b * 2048, 2048)],
                        out_hbm.at[pl.ds(b * (N * 2) + wid * 2048, 2048)])


PLANE_S = K * L


def kernel(vertices, collider, collision_vertices):
    vp = jnp.transpose(vertices, (2, 0, 1)).reshape(3, B, N // 128, 128)
    vp = jnp.transpose(vp, (0, 2, 1, 3)).reshape(-1)
    cp = jnp.transpose(collider, (2, 0, 1)).reshape(3, B, M // 128, 128)
    cp = jnp.transpose(cp, (0, 2, 1, 3)).reshape(-1)

    mesh = plsc.VectorSubcoreMesh(core_axis_name="c", subcore_axis_name="s")
    run = pl.kernel(
        _nn_body,
        out_type=jax.ShapeDtypeStruct((B * N * 2,), jnp.int32),
        mesh=mesh,
        compiler_params=pltpu.CompilerParams(needs_layout_passes=False),
        scratch_types=[
            pltpu.VMEM((QPT,), jnp.float32),
            pltpu.VMEM((QPT,), jnp.float32),
            pltpu.VMEM((QPT,), jnp.float32),
            pltpu.VMEM((K,), jnp.int32),
            pltpu.VMEM((24 * K,), jnp.int32),
            pltpu.VMEM((24 * K,), jnp.float32),
            pltpu.VMEM((3 * B * K * L,), jnp.float32),
            pltpu.VMEM((B * 8 * 2 * 128,), jnp.int32),
            pltpu.SemaphoreType.DMA,
        ],
    )
    x = run(vp, cp, collision_vertices)
    return (x.reshape(B, N // 128, 2, 128)
            .transpose(0, 1, 3, 2)
            .reshape(B, N, 2))
